# Initial kernel scaffold; baseline (speedup 1.0000x reference)
import jax
import jax.numpy as jnp
from jax.experimental import pallas as pl


def kernel(x, W):
    raise NotImplementedError("write your pallas kernel here")


# speedup vs baseline: 1.0951x; 1.0951x over previous
import functools

import jax
import jax.numpy as jnp
from jax import lax
from jax.experimental import pallas as pl
from jax.experimental.pallas import tpu as pltpu
from jax.experimental.pallas import tpu_sc as plsc

VOCAB = 1000000
EMBED = 32
BATCH = 16384
HIST = 50

B = BATCH * HIST
NC, NS = 2, 16
NW = NC * NS
B_PER_W = B // NW
CHUNK = 1024
N_CHUNKS = B_PER_W // CHUNK


def _gather_body(idx_hbm, table_hbm, out_hbm, idx_v, rows_v, sem):
    wid = lax.axis_index("s") * NC + lax.axis_index("c")
    base = wid * B_PER_W

    def step(g, _):
        off = base + g * CHUNK
        pltpu.sync_copy(idx_hbm.at[pl.ds(off, CHUNK)], idx_v)
        pltpu.async_copy(table_hbm.at[idx_v], rows_v, sem).wait()
        pltpu.sync_copy(rows_v, out_hbm.at[pl.ds(off, CHUNK)])
        return 0

    lax.fori_loop(0, N_CHUNKS, step, 0)


@jax.jit
def kernel(x, W):
    idx = x.reshape(-1).astype(jnp.int32)
    mesh = plsc.VectorSubcoreMesh(core_axis_name="c", subcore_axis_name="s")
    out = pl.kernel(
        _gather_body,
        out_type=jax.ShapeDtypeStruct((B, EMBED), jnp.float32),
        mesh=mesh,
        scratch_types=[
            pltpu.VMEM((CHUNK,), jnp.int32),
            pltpu.VMEM((CHUNK, EMBED), jnp.float32),
            pltpu.SemaphoreType.DMA,
        ],
        compiler_params=pltpu.CompilerParams(use_tc_tiling_on_sc=False),
    )(idx, W)
    return out.reshape(BATCH, HIST, EMBED)


# speedup vs baseline: 1.1144x; 1.0177x over previous
import jax
import jax.numpy as jnp
from jax import lax
from jax.experimental import pallas as pl
from jax.experimental.pallas import tpu as pltpu
from jax.experimental.pallas import tpu_sc as plsc

VOCAB = 1000000
EMBED = 32
BATCH = 16384
HIST = 50

B = BATCH * HIST
NC, NS = 2, 16
NW = NC * NS
B_PER_W = B // NW
CHUNK = 1024
N_CHUNKS = B_PER_W // CHUNK
NBUF = 3


def _gather_body(idx_hbm, table_hbm, out_hbm, idx_v, rows_v, sem_g, sem_o):
    wid = lax.axis_index("s") * NC + lax.axis_index("c")
    base = wid * B_PER_W

    pltpu.sync_copy(idx_hbm.at[pl.ds(base, B_PER_W)], idx_v)

    def start_gather(g, s):
        pltpu.async_copy(
            table_hbm.at[idx_v.at[pl.ds(g * CHUNK, CHUNK)]],
            rows_v.at[s],
            sem_g.at[s],
        )

    def start_write(g, s):
        pltpu.async_copy(
            rows_v.at[s],
            out_hbm.at[pl.ds(base + g * CHUNK, CHUNK)],
            sem_o.at[s],
        )

    for s in range(NBUF):
        start_gather(s, s)

    def wait_gather(s):
        pltpu.make_async_copy(
            table_hbm.at[pl.ds(0, CHUNK)], rows_v.at[s], sem_g.at[s]
        ).wait()

    def wait_write(s):
        pltpu.make_async_copy(
            rows_v.at[s], out_hbm.at[pl.ds(0, CHUNK)], sem_o.at[s]
        ).wait()

    def step(g, _):
        s = lax.rem(g, NBUF)
        wait_gather(s)
        start_write(g, s)

        @pl.when(g + NBUF < N_CHUNKS)
        def _():
            wait_write(s)
            start_gather(g + NBUF, s)

        return 0

    lax.fori_loop(0, N_CHUNKS, step, 0)

    for s in range(NBUF):
        wait_write(s)


@jax.jit
def kernel(x, W):
    idx = x.reshape(-1).astype(jnp.int32)
    mesh = plsc.VectorSubcoreMesh(core_axis_name="c", subcore_axis_name="s")
    out = pl.kernel(
        _gather_body,
        out_type=jax.ShapeDtypeStruct((B, EMBED), jnp.float32),
        mesh=mesh,
        scratch_types=[
            pltpu.VMEM((B_PER_W,), jnp.int32),
            pltpu.VMEM((NBUF, CHUNK, EMBED), jnp.float32),
            pltpu.SemaphoreType.DMA((NBUF,)),
            pltpu.SemaphoreType.DMA((NBUF,)),
        ],
        compiler_params=pltpu.CompilerParams(use_tc_tiling_on_sc=False),
    )(idx, W)
    return out.reshape(BATCH, HIST, EMBED)


# speedup vs baseline: 1.8105x; 1.6246x over previous
import jax
import jax.numpy as jnp
from jax import lax
from jax.experimental import pallas as pl
from jax.experimental.pallas import tpu as pltpu
from jax.experimental.pallas import tpu_sc as plsc

VOCAB = 1000000
EMBED = 32
BATCH = 16384
HIST = 50

NC, NS = 2, 16
NW = NC * NS
ROWS_PER_W = BATCH // NW
IDX_PER_W = ROWS_PER_W * HIST
NR = 16
CHUNK = NR * HIST
N_CHUNKS = ROWS_PER_W // NR
NBUF = 3


def _gather_body(idx_hbm, table_hbm, out_hbm, idx_v, rows_v, sem_g, sem_o):
    wid = lax.axis_index("s") * NC + lax.axis_index("c")
    base = wid * ROWS_PER_W

    pltpu.sync_copy(idx_hbm.at[pl.ds(base * HIST, IDX_PER_W)], idx_v)

    def start_gather(g, s):
        pltpu.async_copy(
            table_hbm.at[idx_v.at[pl.ds(g * CHUNK, CHUNK)]],
            rows_v.at[s],
            sem_g.at[s],
        )

    def start_writes(g, s):
        for j in range(NR):
            pltpu.async_copy(
                rows_v.at[s].at[pl.ds(j * HIST, HIST)],
                out_hbm.at[base + g * NR + j],
                sem_o.at[s],
            )

    def wait_gather(s):
        pltpu.make_async_copy(
            table_hbm.at[pl.ds(0, CHUNK)], rows_v.at[s], sem_g.at[s]
        ).wait()

    def wait_writes(s):
        for j in range(NR):
            pltpu.make_async_copy(
                rows_v.at[s].at[pl.ds(0, HIST)], out_hbm.at[0], sem_o.at[s]
            ).wait()

    for s in range(NBUF):
        start_gather(s, s)

    def step(g, _):
        s = lax.rem(g, NBUF)
        wait_gather(s)
        start_writes(g, s)

        @pl.when(g + NBUF < N_CHUNKS)
        def _():
            wait_writes(s)
            start_gather(g + NBUF, s)

        return 0

    lax.fori_loop(0, N_CHUNKS, step, 0)

    for s in range(NBUF):
        wait_writes(s)


@jax.jit
def kernel(x, W):
    idx = x.reshape(-1).astype(jnp.int32)
    mesh = plsc.VectorSubcoreMesh(core_axis_name="c", subcore_axis_name="s")
    return pl.kernel(
        _gather_body,
        out_type=jax.ShapeDtypeStruct((BATCH, HIST, EMBED), jnp.float32),
        mesh=mesh,
        scratch_types=[
            pltpu.VMEM((IDX_PER_W,), jnp.int32),
            pltpu.VMEM((NBUF, CHUNK, EMBED), jnp.float32),
            pltpu.SemaphoreType.DMA((NBUF,)),
            pltpu.SemaphoreType.DMA((NBUF,)),
        ],
        compiler_params=pltpu.CompilerParams(use_tc_tiling_on_sc=False),
    )(idx, W)
